# Initial kernel scaffold; baseline (speedup 1.0000x reference)
import jax
import jax.numpy as jnp
from jax.experimental import pallas as pl


def kernel(x, W1, b1, W2, b2, W3, b3, Wg, bg):
    raise NotImplementedError("write your pallas kernel here")


# speedup vs baseline: 1.6247x; 1.6247x over previous
import functools

import jax
import jax.numpy as jnp
from jax import lax
from jax.experimental import pallas as pl
from jax.experimental.pallas import tpu as pltpu

_B, _S, _D = 4, 2048, 1024
_H = 2048
_E = 8
_N = _B * _S
_NP = _N * 2
_T = 256
_NT = _NP // _T + _E
_NS = _NT * _T
_GB = 1024


def _gating_body(x_ref, wg_ref, bg_ref, i0_ref, i1_ref, g0_ref, g1_ref,
                 psum_ref, csum_ref):
    i = pl.program_id(0)
    xb = x_ref[...]
    logits = jnp.dot(xb, wg_ref[...], preferred_element_type=jnp.float32)
    logits = logits + bg_ref[...]
    e_iota = lax.broadcasted_iota(jnp.int32, (_GB, _E), 1)
    l0 = jnp.max(logits, axis=1, keepdims=True)
    i0 = jnp.min(jnp.where(logits == l0, e_iota, _E), axis=1, keepdims=True)
    masked = jnp.where(e_iota == i0, -1e30, logits)
    l1 = jnp.max(masked, axis=1, keepdims=True)
    i1 = jnp.min(jnp.where(masked == l1, e_iota, _E), axis=1, keepdims=True)
    t = jnp.exp(l1 - l0)
    g0 = 1.0 / (1.0 + t)
    g1 = t / (1.0 + t)
    i0_ref[...] = i0.reshape(1, 1, _GB)
    i1_ref[...] = i1.reshape(1, 1, _GB)
    g0_ref[...] = g0.reshape(1, 1, _GB)
    g1_ref[...] = g1.reshape(1, 1, _GB)
    p = jnp.exp(logits - l0)
    p = p / jnp.sum(p, axis=1, keepdims=True)
    psum = jnp.sum(p, axis=0, keepdims=True)
    oh0 = (e_iota == i0).astype(jnp.float32)
    oh1 = (e_iota == i1).astype(jnp.float32)
    csum = jnp.sum(g0 * oh0 + g1 * oh1, axis=0, keepdims=True)

    @pl.when(i == 0)
    def _():
        psum_ref[...] = jnp.zeros_like(psum_ref)
        csum_ref[...] = jnp.zeros_like(csum_ref)

    psum_ref[...] += psum
    csum_ref[...] += csum


def _gating(x2d, Wg, bg):
    nb = _N // _GB
    out = pl.pallas_call(
        _gating_body,
        grid=(nb,),
        in_specs=[
            pl.BlockSpec((_GB, _D), lambda i: (i, 0)),
            pl.BlockSpec((_D, _E), lambda i: (0, 0)),
            pl.BlockSpec((1, _E), lambda i: (0, 0)),
        ],
        out_specs=[
            pl.BlockSpec((1, 1, _GB), lambda i: (i, 0, 0)),
            pl.BlockSpec((1, 1, _GB), lambda i: (i, 0, 0)),
            pl.BlockSpec((1, 1, _GB), lambda i: (i, 0, 0)),
            pl.BlockSpec((1, 1, _GB), lambda i: (i, 0, 0)),
            pl.BlockSpec((1, _E), lambda i: (0, 0)),
            pl.BlockSpec((1, _E), lambda i: (0, 0)),
        ],
        out_shape=[
            jax.ShapeDtypeStruct((nb, 1, _GB), jnp.int32),
            jax.ShapeDtypeStruct((nb, 1, _GB), jnp.int32),
            jax.ShapeDtypeStruct((nb, 1, _GB), jnp.float32),
            jax.ShapeDtypeStruct((nb, 1, _GB), jnp.float32),
            jax.ShapeDtypeStruct((1, _E), jnp.float32),
            jax.ShapeDtypeStruct((1, _E), jnp.float32),
        ],
        compiler_params=pltpu.CompilerParams(
            dimension_semantics=("arbitrary",)),
    )(x2d, Wg, bg.reshape(1, _E))
    i0, i1, g0, g1, psum, csum = out
    return (i0.reshape(_N), i1.reshape(_N), g0.reshape(_N), g1.reshape(_N),
            psum.reshape(_E), csum.reshape(_E))


def _mm_body(te_ref, x_ref, w_ref, b_ref, o_ref, *, act):
    acc = jnp.dot(x_ref[...], w_ref[0], preferred_element_type=jnp.float32)
    acc = acc + b_ref[0]
    if act:
        acc = jnp.maximum(acc, 0.0)
    o_ref[...] = acc


def _grouped_mm(te, xs, W, b, act):
    K, M = W.shape[1], W.shape[2]
    return pl.pallas_call(
        functools.partial(_mm_body, act=act),
        grid_spec=pltpu.PrefetchScalarGridSpec(
            num_scalar_prefetch=1,
            grid=(_NT,),
            in_specs=[
                pl.BlockSpec((_T, K), lambda t, s: (t, 0)),
                pl.BlockSpec((1, K, M), lambda t, s: (s[t], 0, 0)),
                pl.BlockSpec((1, 1, M), lambda t, s: (s[t], 0, 0)),
            ],
            out_specs=pl.BlockSpec((_T, M), lambda t, s: (t, 0)),
        ),
        out_shape=jax.ShapeDtypeStruct((_NS, M), jnp.float32),
        compiler_params=pltpu.CompilerParams(
            dimension_semantics=("arbitrary",)),
    )(te, xs, W, b.reshape(_E, 1, M))


def kernel(x, W1, b1, W2, b2, W3, b3, Wg, bg):
    x2d = x.reshape(_N, _D)
    i0, i1, g0, g1, psum, csum = _gating(x2d, Wg, bg)

    e_pairs = jnp.stack([i0, i1], axis=1).reshape(_NP)
    onehot = (e_pairs[:, None] == jnp.arange(_E)[None, :])
    counts = jnp.sum(onehot, axis=0, dtype=jnp.int32)
    padded = ((counts + _T - 1) // _T) * _T
    base = jnp.concatenate([jnp.zeros(1, jnp.int32),
                            jnp.cumsum(padded)[:-1].astype(jnp.int32)])
    rank = jnp.cumsum(onehot, axis=0, dtype=jnp.int32) - onehot.astype(jnp.int32)
    pos = base[e_pairs] + jnp.sum(jnp.where(onehot, rank, 0), axis=1)
    ends = base + padded
    tile_start = jnp.arange(_NT, dtype=jnp.int32) * _T
    te = jnp.minimum(jnp.sum(tile_start[:, None] >= ends[None, :], axis=1),
                     _E - 1).astype(jnp.int32)

    slot_tok = jnp.zeros((_NS,), jnp.int32).at[pos].set(
        jnp.arange(_NP, dtype=jnp.int32) // 2)
    xs = x2d[slot_tok]

    h = _grouped_mm(te, xs, W1, b1, act=True)
    h = _grouped_mm(te, h, W2, b2, act=True)
    y = _grouped_mm(te, h, W3, b3, act=False)

    pp = pos.reshape(_N, 2)
    out2d = g0[:, None] * y[pp[:, 0]] + g1[:, None] * y[pp[:, 1]]

    avg_probs = psum / _N
    avg_counts = csum / _N
    lb = 0.01 * _E * jnp.sum(avg_probs * avg_counts)
    ent = -jnp.sum(avg_probs * jnp.log(avg_probs + 1e-08))
    return (out2d.reshape(_B, _S, _D), lb, avg_counts, ent)


# speedup vs baseline: 1.9049x; 1.1724x over previous
import functools

import jax
import jax.numpy as jnp
from jax import lax
from jax.experimental import pallas as pl
from jax.experimental.pallas import tpu as pltpu
from jax.experimental.pallas import tpu_sc as plsc

_B, _S, _D = 4, 2048, 1024
_H = 2048
_E = 8
_N = _B * _S
_NP = _N * 2
_T = 256
_NT = _NP // _T + _E
_NS = _NT * _T
_GB = 1024

_NW = 32
_CHUNK = _NP // _NW
_CV = _CHUNK // 16
_NTP = 80
_TK = _N // _NW
_CC = 32


def _gating_body(x_ref, wg_ref, bg_ref, i0_ref, i1_ref, g0_ref, g1_ref,
                 psum_ref, csum_ref, kcnt_ref):
    i = pl.program_id(0)
    xb = x_ref[...]
    logits = jnp.dot(xb, wg_ref[...], preferred_element_type=jnp.float32)
    logits = logits + bg_ref[...]
    e_iota = lax.broadcasted_iota(jnp.int32, (_GB, _E), 1)
    l0 = jnp.max(logits, axis=1, keepdims=True)
    i0 = jnp.min(jnp.where(logits == l0, e_iota, _E), axis=1, keepdims=True)
    masked = jnp.where(e_iota == i0, -1e30, logits)
    l1 = jnp.max(masked, axis=1, keepdims=True)
    i1 = jnp.min(jnp.where(masked == l1, e_iota, _E), axis=1, keepdims=True)
    t = jnp.exp(l1 - l0)
    g0 = 1.0 / (1.0 + t)
    g1 = t / (1.0 + t)
    i0_ref[...] = i0.reshape(1, 1, _GB)
    i1_ref[...] = i1.reshape(1, 1, _GB)
    g0_ref[...] = g0.reshape(1, 1, _GB)
    g1_ref[...] = g1.reshape(1, 1, _GB)
    p = jnp.exp(logits - l0)
    p = p / jnp.sum(p, axis=1, keepdims=True)
    psum = jnp.sum(p, axis=0, keepdims=True)
    oh0 = (e_iota == i0).astype(jnp.float32)
    oh1 = (e_iota == i1).astype(jnp.float32)
    csum = jnp.sum(g0 * oh0 + g1 * oh1, axis=0, keepdims=True)
    ksum = jnp.sum((oh0 + oh1).astype(jnp.int32), axis=0, keepdims=True)

    @pl.when(i == 0)
    def _():
        psum_ref[...] = jnp.zeros_like(psum_ref)
        csum_ref[...] = jnp.zeros_like(csum_ref)
        kcnt_ref[...] = jnp.zeros_like(kcnt_ref)

    psum_ref[...] += psum
    csum_ref[...] += csum
    kcnt_ref[...] += ksum


def _gating(x2d, Wg, bg):
    nb = _N // _GB
    out = pl.pallas_call(
        _gating_body,
        grid=(nb,),
        in_specs=[
            pl.BlockSpec((_GB, _D), lambda i: (i, 0)),
            pl.BlockSpec((_D, _E), lambda i: (0, 0)),
            pl.BlockSpec((1, _E), lambda i: (0, 0)),
        ],
        out_specs=[
            pl.BlockSpec((1, 1, _GB), lambda i: (i, 0, 0)),
            pl.BlockSpec((1, 1, _GB), lambda i: (i, 0, 0)),
            pl.BlockSpec((1, 1, _GB), lambda i: (i, 0, 0)),
            pl.BlockSpec((1, 1, _GB), lambda i: (i, 0, 0)),
            pl.BlockSpec((1, _E), lambda i: (0, 0)),
            pl.BlockSpec((1, _E), lambda i: (0, 0)),
            pl.BlockSpec((1, _E), lambda i: (0, 0)),
        ],
        out_shape=[
            jax.ShapeDtypeStruct((nb, 1, _GB), jnp.int32),
            jax.ShapeDtypeStruct((nb, 1, _GB), jnp.int32),
            jax.ShapeDtypeStruct((nb, 1, _GB), jnp.float32),
            jax.ShapeDtypeStruct((nb, 1, _GB), jnp.float32),
            jax.ShapeDtypeStruct((1, _E), jnp.float32),
            jax.ShapeDtypeStruct((1, _E), jnp.float32),
            jax.ShapeDtypeStruct((1, _E), jnp.int32),
        ],
        compiler_params=pltpu.CompilerParams(
            dimension_semantics=("arbitrary",)),
    )(x2d, Wg, bg.reshape(1, _E))
    i0, i1, g0, g1, psum, csum, kcnt = out
    return (i0.reshape(_N), i1.reshape(_N), g0.reshape(_N), g1.reshape(_N),
            psum.reshape(_E), csum.reshape(_E), kcnt.reshape(_E))


def _take16(vec, idx):
    return lax.gather(
        vec, idx[:, None],
        lax.GatherDimensionNumbers(offset_dims=(), collapsed_slice_dims=(0,),
                                   start_index_map=(0,)),
        slice_sizes=(1,),
        mode=lax.GatherScatterMode.PROMISE_IN_BOUNDS)


def _route_body(ids_hbm, cnt_hbm, x_hbm, xs_hbm, pos_hbm, te_hbm,
                ids_v, cnt_v, pos_v, posidx_v, xrow_v, te_v, sem):
    wid = lax.axis_index("s") * 2 + lax.axis_index("c")
    pltpu.sync_copy(ids_hbm, ids_v)
    pltpu.sync_copy(cnt_hbm, cnt_v)
    zeros16 = jnp.zeros((16,), jnp.int32)
    lanes = lax.iota(jnp.int32, 16)

    cnt = cnt_v[...]
    pc = ((cnt + (_T - 1)) >> 8) << 8
    bases = plsc.cumsum(pc) - pc
    ends = bases + pc

    base_j = wid * _CV

    def count_step(j, pre):
        ev = ids_v[pl.ds(j * 16, 16)]
        for e in range(_E):
            c = jnp.sum((ev == e).astype(jnp.int32))
            pre = pre + jnp.where(lanes == e, c, 0)
        return pre

    pre = lax.fori_loop(0, base_j, count_step, zeros16)

    tok0 = (wid * _CHUNK) % _N

    def pos_step(v, start):
        ev = ids_v[pl.ds((base_j + v) * 16, 16)]
        sv = _take16(start, ev)
        r = zeros16
        delta = zeros16
        for e in range(_E):
            m = ev == e
            mi = m.astype(jnp.int32)
            cs = plsc.cumsum(mi)
            r = jnp.where(m, cs, r)
            delta = delta + jnp.where(lanes == e, jnp.sum(mi), 0)
        pvec = sv + r - 1
        pos_v[pl.ds(v * 16, 16)] = pvec
        posidx_v[...] = pvec
        pltpu.sync_copy(x_hbm.at[pl.ds(tok0 + v * 16, 16)], xrow_v)
        pltpu.async_copy(xrow_v, xs_hbm.at[posidx_v], sem).wait()
        return start + delta

    lax.fori_loop(0, _CV, pos_step, bases + pre)
    pltpu.sync_copy(pos_v, pos_hbm.at[pl.ds(wid * _CHUNK, _CHUNK)])

    @pl.when(wid == 0)
    def _():
        for j in range(_NTP // 16):
            tv = (lanes + j * 16) * _T
            acc = zeros16
            for e in range(_E):
                end_e = _take16(ends, jnp.full((16,), e, jnp.int32))
                acc += (tv >= end_e).astype(jnp.int32)
            te_v[pl.ds(j * 16, 16)] = jnp.minimum(acc, _E - 1)
        pltpu.sync_copy(te_v, te_hbm)


def _route(ids, counts16, x2d):
    f = pl.kernel(
        _route_body,
        mesh=plsc.VectorSubcoreMesh(core_axis_name="c", subcore_axis_name="s"),
        out_type=[
            jax.ShapeDtypeStruct((_NS, _D), jnp.float32),
            jax.ShapeDtypeStruct((_NP,), jnp.int32),
            jax.ShapeDtypeStruct((_NTP,), jnp.int32),
        ],
        scratch_types=[
            pltpu.VMEM((_NP,), jnp.int32),
            pltpu.VMEM((16,), jnp.int32),
            pltpu.VMEM((_CHUNK,), jnp.int32),
            pltpu.VMEM((16,), jnp.int32),
            pltpu.VMEM((16, _D), jnp.float32),
            pltpu.VMEM((_NTP,), jnp.int32),
            pltpu.SemaphoreType.DMA,
        ],
        compiler_params=pltpu.CompilerParams(needs_layout_passes=False),
    )
    return f(ids, counts16, x2d)


def _mm_body(te_ref, x_ref, w_ref, b_ref, o_ref, *, act):
    acc = jnp.dot(x_ref[...], w_ref[0], preferred_element_type=jnp.float32)
    acc = acc + b_ref[0]
    if act:
        acc = jnp.maximum(acc, 0.0)
    o_ref[...] = acc


def _grouped_mm(te, xs, W, b, act):
    K, M = W.shape[1], W.shape[2]
    return pl.pallas_call(
        functools.partial(_mm_body, act=act),
        grid_spec=pltpu.PrefetchScalarGridSpec(
            num_scalar_prefetch=1,
            grid=(_NT,),
            in_specs=[
                pl.BlockSpec((_T, K), lambda t, s: (t, 0)),
                pl.BlockSpec((1, K, M), lambda t, s: (s[t], 0, 0)),
                pl.BlockSpec((1, 1, M), lambda t, s: (s[t], 0, 0)),
            ],
            out_specs=pl.BlockSpec((_T, M), lambda t, s: (t, 0)),
        ),
        out_shape=jax.ShapeDtypeStruct((_NS, M), jnp.float32),
        compiler_params=pltpu.CompilerParams(
            dimension_semantics=("arbitrary",)),
    )(te, xs, W, b.reshape(_E, 1, M))


def _combine_body(y_hbm, pos_hbm, g0_hbm, g1_hbm, out_hbm,
                  p0_v, p1_v, g0_v, g1_v, a_v, b_v, o_v, sem):
    wid = lax.axis_index("s") * 2 + lax.axis_index("c")
    t0 = wid * _TK
    pltpu.sync_copy(pos_hbm.at[pl.ds(t0, _TK)], p0_v)
    pltpu.sync_copy(pos_hbm.at[pl.ds(_N + t0, _TK)], p1_v)
    pltpu.sync_copy(g0_hbm.at[pl.ds(t0, _TK)], g0_v)
    pltpu.sync_copy(g1_hbm.at[pl.ds(t0, _TK)], g1_v)

    def chunk_step(cj, carry):
        pltpu.async_copy(
            y_hbm.at[p0_v.at[pl.ds(cj * _CC, _CC)]], a_v, sem).wait()
        pltpu.async_copy(
            y_hbm.at[p1_v.at[pl.ds(cj * _CC, _CC)]], b_v, sem).wait()

        def tok_step(j, carry2):
            grp = cj * _CC + (j // 16) * 16
            lane = jnp.full((16,), j % 16, jnp.int32)
            g0s = _take16(g0_v[pl.ds(grp, 16)], lane)
            g1s = _take16(g1_v[pl.ds(grp, 16)], lane)
            for seg in range(_D // 16):
                av = a_v[j, pl.ds(seg * 16, 16)]
                bv = b_v[j, pl.ds(seg * 16, 16)]
                o_v[j, pl.ds(seg * 16, 16)] = g0s * av + g1s * bv
            return carry2

        lax.fori_loop(0, _CC, tok_step, 0)
        pltpu.sync_copy(o_v, out_hbm.at[pl.ds(t0 + cj * _CC, _CC)])
        return carry

    lax.fori_loop(0, _TK // _CC, chunk_step, 0)


def _combine(y, pos, g0, g1):
    f = pl.kernel(
        _combine_body,
        mesh=plsc.VectorSubcoreMesh(core_axis_name="c", subcore_axis_name="s"),
        out_type=jax.ShapeDtypeStruct((_N, _D), jnp.float32),
        scratch_types=[
            pltpu.VMEM((_TK,), jnp.int32),
            pltpu.VMEM((_TK,), jnp.int32),
            pltpu.VMEM((_TK,), jnp.float32),
            pltpu.VMEM((_TK,), jnp.float32),
            pltpu.VMEM((_CC, _D), jnp.float32),
            pltpu.VMEM((_CC, _D), jnp.float32),
            pltpu.VMEM((_CC, _D), jnp.float32),
            pltpu.SemaphoreType.DMA,
        ],
        compiler_params=pltpu.CompilerParams(needs_layout_passes=False),
    )
    return f(y, pos, g0, g1)


def kernel(x, W1, b1, W2, b2, W3, b3, Wg, bg):
    x2d = x.reshape(_N, _D)
    i0, i1, g0, g1, psum, csum, kcnt = _gating(x2d, Wg, bg)

    ids = jnp.concatenate([i0, i1])
    counts16 = jnp.concatenate([kcnt, jnp.zeros((16 - _E,), jnp.int32)])
    xs, pos, te_pad = _route(ids, counts16, x2d)
    te = te_pad[:_NT]

    h = _grouped_mm(te, xs, W1, b1, act=True)
    h = _grouped_mm(te, h, W2, b2, act=True)
    y = _grouped_mm(te, h, W3, b3, act=False)

    out2d = _combine(y, pos, g0, g1)

    avg_probs = psum / _N
    avg_counts = csum / _N
    lb = 0.01 * _E * jnp.sum(avg_probs * avg_counts)
    ent = -jnp.sum(avg_probs * jnp.log(avg_probs + 1e-08))
    return (out2d.reshape(_B, _S, _D), lb, avg_counts, ent)


# speedup vs baseline: 2.0895x; 1.0969x over previous
import functools

import jax
import jax.numpy as jnp
from jax import lax
from jax.experimental import pallas as pl
from jax.experimental.pallas import tpu as pltpu
from jax.experimental.pallas import tpu_sc as plsc

_B, _S, _D = 4, 2048, 1024
_H = 2048
_E = 8
_N = _B * _S
_NP = _N * 2
_T = 256
_NT = _NP // _T + _E
_NS = _NT * _T
_GB = 1024

_NW = 32
_CHUNK = _NP // _NW
_CV = _CHUNK // 16
_NTP = 80
_TK = _N // _NW
_CC = 32


def _gating_body(x_ref, wg_ref, bg_ref, i0_ref, i1_ref, g0_ref, g1_ref,
                 psum_ref, csum_ref, kcnt_ref):
    i = pl.program_id(0)
    xb = x_ref[...]
    logits = jnp.dot(xb, wg_ref[...], preferred_element_type=jnp.float32)
    logits = logits + bg_ref[...]
    e_iota = lax.broadcasted_iota(jnp.int32, (_GB, _E), 1)
    l0 = jnp.max(logits, axis=1, keepdims=True)
    i0 = jnp.min(jnp.where(logits == l0, e_iota, _E), axis=1, keepdims=True)
    masked = jnp.where(e_iota == i0, -1e30, logits)
    l1 = jnp.max(masked, axis=1, keepdims=True)
    i1 = jnp.min(jnp.where(masked == l1, e_iota, _E), axis=1, keepdims=True)
    t = jnp.exp(l1 - l0)
    g0 = 1.0 / (1.0 + t)
    g1 = t / (1.0 + t)
    i0_ref[...] = i0.reshape(1, 1, _GB)
    i1_ref[...] = i1.reshape(1, 1, _GB)
    g0_ref[...] = g0.reshape(1, 1, _GB)
    g1_ref[...] = g1.reshape(1, 1, _GB)
    p = jnp.exp(logits - l0)
    p = p / jnp.sum(p, axis=1, keepdims=True)
    psum = jnp.sum(p, axis=0, keepdims=True)
    oh0 = (e_iota == i0).astype(jnp.float32)
    oh1 = (e_iota == i1).astype(jnp.float32)
    csum = jnp.sum(g0 * oh0 + g1 * oh1, axis=0, keepdims=True)
    ksum = jnp.sum((oh0 + oh1).astype(jnp.int32), axis=0, keepdims=True)

    @pl.when(i == 0)
    def _():
        psum_ref[...] = jnp.zeros_like(psum_ref)
        csum_ref[...] = jnp.zeros_like(csum_ref)
        kcnt_ref[...] = jnp.zeros_like(kcnt_ref)

    psum_ref[...] += psum
    csum_ref[...] += csum
    kcnt_ref[...] += ksum


def _gating(x2d, Wg, bg):
    nb = _N // _GB
    out = pl.pallas_call(
        _gating_body,
        grid=(nb,),
        in_specs=[
            pl.BlockSpec((_GB, _D), lambda i: (i, 0)),
            pl.BlockSpec((_D, _E), lambda i: (0, 0)),
            pl.BlockSpec((1, _E), lambda i: (0, 0)),
        ],
        out_specs=[
            pl.BlockSpec((1, 1, _GB), lambda i: (i, 0, 0)),
            pl.BlockSpec((1, 1, _GB), lambda i: (i, 0, 0)),
            pl.BlockSpec((1, 1, _GB), lambda i: (i, 0, 0)),
            pl.BlockSpec((1, 1, _GB), lambda i: (i, 0, 0)),
            pl.BlockSpec((1, _E), lambda i: (0, 0)),
            pl.BlockSpec((1, _E), lambda i: (0, 0)),
            pl.BlockSpec((1, _E), lambda i: (0, 0)),
        ],
        out_shape=[
            jax.ShapeDtypeStruct((nb, 1, _GB), jnp.int32),
            jax.ShapeDtypeStruct((nb, 1, _GB), jnp.int32),
            jax.ShapeDtypeStruct((nb, 1, _GB), jnp.float32),
            jax.ShapeDtypeStruct((nb, 1, _GB), jnp.float32),
            jax.ShapeDtypeStruct((1, _E), jnp.float32),
            jax.ShapeDtypeStruct((1, _E), jnp.float32),
            jax.ShapeDtypeStruct((1, _E), jnp.int32),
        ],
        compiler_params=pltpu.CompilerParams(
            dimension_semantics=("arbitrary",)),
    )(x2d, Wg, bg.reshape(1, _E))
    i0, i1, g0, g1, psum, csum, kcnt = out
    return (i0.reshape(_N), i1.reshape(_N), g0.reshape(_N), g1.reshape(_N),
            psum.reshape(_E), csum.reshape(_E), kcnt.reshape(_E))


def _take16(vec, idx):
    return lax.gather(
        vec, idx[:, None],
        lax.GatherDimensionNumbers(offset_dims=(), collapsed_slice_dims=(0,),
                                   start_index_map=(0,)),
        slice_sizes=(1,),
        mode=lax.GatherScatterMode.PROMISE_IN_BOUNDS)


def _route_body(ids_hbm, cnt_hbm, x_hbm, xs_hbm, pos_hbm, te_hbm,
                ids_v, cnt_v, pos_v, posidx_v, xrow_v, te_v, sem):
    wid = lax.axis_index("s") * 2 + lax.axis_index("c")
    pltpu.sync_copy(ids_hbm, ids_v)
    pltpu.sync_copy(cnt_hbm, cnt_v)
    zeros16 = jnp.zeros((16,), jnp.int32)
    lanes = lax.iota(jnp.int32, 16)

    cnt = cnt_v[...]
    pc = ((cnt + (_T - 1)) >> 8) << 8
    bases = plsc.cumsum(pc) - pc
    ends = bases + pc

    base_j = wid * _CV

    def count_step(j, pre):
        ev = ids_v[pl.ds(j * 16, 16)]
        for e in range(_E):
            c = jnp.sum((ev == e).astype(jnp.int32))
            pre = pre + jnp.where(lanes == e, c, 0)
        return pre

    pre = lax.fori_loop(0, base_j, count_step, zeros16)

    tok0 = (wid * _CHUNK) % _N

    def pos_step(v, start):
        ev = ids_v[pl.ds((base_j + v) * 16, 16)]
        sv = _take16(start, ev)
        r = zeros16
        delta = zeros16
        for e in range(_E):
            m = ev == e
            mi = m.astype(jnp.int32)
            cs = plsc.cumsum(mi)
            r = jnp.where(m, cs, r)
            delta = delta + jnp.where(lanes == e, jnp.sum(mi), 0)
        pvec = sv + r - 1
        pos_v[pl.ds(v * 16, 16)] = pvec
        posidx_v[...] = pvec
        pltpu.sync_copy(x_hbm.at[pl.ds(tok0 + v * 16, 16)], xrow_v)
        pltpu.async_copy(xrow_v, xs_hbm.at[posidx_v], sem).wait()
        return start + delta

    lax.fori_loop(0, _CV, pos_step, bases + pre)
    pltpu.sync_copy(pos_v, pos_hbm.at[pl.ds(wid * _CHUNK, _CHUNK)])

    @pl.when(wid == 0)
    def _():
        for j in range(_NTP // 16):
            tv = (lanes + j * 16) * _T
            acc = zeros16
            for e in range(_E):
                end_e = _take16(ends, jnp.full((16,), e, jnp.int32))
                acc += (tv >= end_e).astype(jnp.int32)
            te_v[pl.ds(j * 16, 16)] = acc
        pltpu.sync_copy(te_v, te_hbm)


def _route(ids, counts16, x2d):
    f = pl.kernel(
        _route_body,
        mesh=plsc.VectorSubcoreMesh(core_axis_name="c", subcore_axis_name="s"),
        out_type=[
            jax.ShapeDtypeStruct((_NS, _D), jnp.float32),
            jax.ShapeDtypeStruct((_NP,), jnp.int32),
            jax.ShapeDtypeStruct((_NTP,), jnp.int32),
        ],
        scratch_types=[
            pltpu.VMEM((_NP,), jnp.int32),
            pltpu.VMEM((16,), jnp.int32),
            pltpu.VMEM((_CHUNK,), jnp.int32),
            pltpu.VMEM((16,), jnp.int32),
            pltpu.VMEM((16, _D), jnp.float32),
            pltpu.VMEM((_NTP,), jnp.int32),
            pltpu.SemaphoreType.DMA,
        ],
        compiler_params=pltpu.CompilerParams(needs_layout_passes=False),
    )
    return f(ids, counts16, x2d)


def _clampe(i):
    return jnp.minimum(i, _E - 1)


def _mm12_body(te_ref, x_ref, w1_ref, b1_ref, w2_ref, b2_ref, o_ref):
    t = pl.program_id(0)

    @pl.when(te_ref[t] < _E)
    def _():
        h1 = jnp.dot(x_ref[...], w1_ref[0],
                     preferred_element_type=jnp.float32)
        h1 = jnp.maximum(h1 + b1_ref[0], 0.0)
        h2 = jnp.dot(h1, w2_ref[0], preferred_element_type=jnp.float32)
        o_ref[...] = jnp.maximum(h2 + b2_ref[0], 0.0)


def _grouped_mm12(te, xs, W1, b1, W2, b2):
    return pl.pallas_call(
        _mm12_body,
        grid_spec=pltpu.PrefetchScalarGridSpec(
            num_scalar_prefetch=1,
            grid=(_NT,),
            in_specs=[
                pl.BlockSpec((_T, _D), lambda t, s: (t, 0)),
                pl.BlockSpec((1, _D, _H), lambda t, s: (_clampe(s[t]), 0, 0)),
                pl.BlockSpec((1, 1, _H), lambda t, s: (_clampe(s[t]), 0, 0)),
                pl.BlockSpec((1, _H, _H), lambda t, s: (_clampe(s[t]), 0, 0)),
                pl.BlockSpec((1, 1, _H), lambda t, s: (_clampe(s[t]), 0, 0)),
            ],
            out_specs=pl.BlockSpec((_T, _H), lambda t, s: (t, 0)),
        ),
        out_shape=jax.ShapeDtypeStruct((_NS, _H), jnp.float32),
        compiler_params=pltpu.CompilerParams(
            dimension_semantics=("arbitrary",)),
    )(te, xs, W1, b1.reshape(_E, 1, _H), W2, b2.reshape(_E, 1, _H))


def _mm3_body(te_ref, x_ref, w_ref, b_ref, o_ref):
    t = pl.program_id(0)

    @pl.when(te_ref[t] < _E)
    def _():
        acc = jnp.dot(x_ref[...], w_ref[0],
                      preferred_element_type=jnp.float32)
        o_ref[...] = acc + b_ref[0]


def _grouped_mm3(te, h, W3, b3):
    return pl.pallas_call(
        _mm3_body,
        grid_spec=pltpu.PrefetchScalarGridSpec(
            num_scalar_prefetch=1,
            grid=(_NT,),
            in_specs=[
                pl.BlockSpec((_T, _H), lambda t, s: (t, 0)),
                pl.BlockSpec((1, _H, _D), lambda t, s: (_clampe(s[t]), 0, 0)),
                pl.BlockSpec((1, 1, _D), lambda t, s: (_clampe(s[t]), 0, 0)),
            ],
            out_specs=pl.BlockSpec((_T, _D), lambda t, s: (t, 0)),
        ),
        out_shape=jax.ShapeDtypeStruct((_NS, _D), jnp.float32),
        compiler_params=pltpu.CompilerParams(
            dimension_semantics=("arbitrary",)),
    )(te, h, W3, b3.reshape(_E, 1, _D))


def _combine_body(y_hbm, pos_hbm, g0_hbm, g1_hbm, out_hbm,
                  p0_v, p1_v, g0_v, g1_v, a_v, b_v, o_v, sem):
    wid = lax.axis_index("s") * 2 + lax.axis_index("c")
    t0 = wid * _TK
    pltpu.sync_copy(pos_hbm.at[pl.ds(t0, _TK)], p0_v)
    pltpu.sync_copy(pos_hbm.at[pl.ds(_N + t0, _TK)], p1_v)
    pltpu.sync_copy(g0_hbm.at[pl.ds(t0, _TK)], g0_v)
    pltpu.sync_copy(g1_hbm.at[pl.ds(t0, _TK)], g1_v)

    def chunk_step(cj, carry):
        pltpu.async_copy(
            y_hbm.at[p0_v.at[pl.ds(cj * _CC, _CC)]], a_v, sem).wait()
        pltpu.async_copy(
            y_hbm.at[p1_v.at[pl.ds(cj * _CC, _CC)]], b_v, sem).wait()

        def tok_step(j, carry2):
            grp = cj * _CC + (j // 16) * 16
            lane = jnp.full((16,), j % 16, jnp.int32)
            g0s = _take16(g0_v[pl.ds(grp, 16)], lane)
            g1s = _take16(g1_v[pl.ds(grp, 16)], lane)
            for seg in range(_D // 16):
                av = a_v[j, pl.ds(seg * 16, 16)]
                bv = b_v[j, pl.ds(seg * 16, 16)]
                o_v[j, pl.ds(seg * 16, 16)] = g0s * av + g1s * bv
            return carry2

        lax.fori_loop(0, _CC, tok_step, 0)
        pltpu.sync_copy(o_v, out_hbm.at[pl.ds(t0 + cj * _CC, _CC)])
        return carry

    lax.fori_loop(0, _TK // _CC, chunk_step, 0)


def _combine(y, pos, g0, g1):
    f = pl.kernel(
        _combine_body,
        mesh=plsc.VectorSubcoreMesh(core_axis_name="c", subcore_axis_name="s"),
        out_type=jax.ShapeDtypeStruct((_N, _D), jnp.float32),
        scratch_types=[
            pltpu.VMEM((_TK,), jnp.int32),
            pltpu.VMEM((_TK,), jnp.int32),
            pltpu.VMEM((_TK,), jnp.float32),
            pltpu.VMEM((_TK,), jnp.float32),
            pltpu.VMEM((_CC, _D), jnp.float32),
            pltpu.VMEM((_CC, _D), jnp.float32),
            pltpu.VMEM((_CC, _D), jnp.float32),
            pltpu.SemaphoreType.DMA,
        ],
        compiler_params=pltpu.CompilerParams(needs_layout_passes=False),
    )
    return f(y, pos, g0, g1)


def kernel(x, W1, b1, W2, b2, W3, b3, Wg, bg):
    x2d = x.reshape(_N, _D)
    i0, i1, g0, g1, psum, csum, kcnt = _gating(x2d, Wg, bg)

    ids = jnp.concatenate([i0, i1])
    counts16 = jnp.concatenate([kcnt, jnp.zeros((16 - _E,), jnp.int32)])
    xs, pos, te_pad = _route(ids, counts16, x2d)
    te = te_pad[:_NT]

    h = _grouped_mm12(te, xs, W1, b1, W2, b2)
    y = _grouped_mm3(te, h, W3, b3)

    out2d = _combine(y, pos, g0, g1)

    avg_probs = psum / _N
    avg_counts = csum / _N
    lb = 0.01 * _E * jnp.sum(avg_probs * avg_counts)
    ent = -jnp.sum(avg_probs * jnp.log(avg_probs + 1e-08))
    return (out2d.reshape(_B, _S, _D), lb, avg_counts, ent)
